# initial kernel scaffold (unmeasured)
import jax
import jax.numpy as jnp
from jax import lax
from jax.experimental import pallas as pl
from jax.experimental.pallas import tpu as pltpu

NX = 2
CHUNK = 512


def kernel(x, assign, W1, W2):
    T, D = x.shape
    E, _, F = W1.shape
    TT = NX * T

    xb = x.astype(jnp.bfloat16)
    a2 = assign.reshape(T, 1)
    w1b = W1.astype(jnp.bfloat16)
    w2b = W2.astype(jnp.bfloat16)

    def body(x_ref, a_ref, w1_hbm, w2_hbm, out_ref,
             xall, aall, acc, resbuf, w1v, w2v,
             sem_sx, sem_rx, sem_sa, sem_ra, sem_sr, sem_rr,
             w1sem, w2sem):
        my_x = lax.axis_index("x")
        my_y = lax.axis_index("y")
        my_z = lax.axis_index("z")
        peer = (1 - my_x, my_y, my_z)

        wcp = [
            (pltpu.make_async_copy(w1_hbm.at[e], w1v.at[e % 2], w1sem.at[e % 2]),
             pltpu.make_async_copy(w2_hbm.at[e], w2v.at[e % 2], w2sem.at[e % 2]))
            for e in range(E)
        ]
        wcp[0][0].start()
        wcp[0][1].start()

        barrier = pltpu.get_barrier_semaphore()
        pl.semaphore_signal(barrier, inc=1, device_id=peer,
                            device_id_type=pl.DeviceIdType.MESH)
        pl.semaphore_wait(barrier, 1)

        my_off = my_x * T
        rx = pltpu.make_async_remote_copy(
            src_ref=x_ref, dst_ref=xall.at[pl.ds(my_off, T)],
            send_sem=sem_sx, recv_sem=sem_rx,
            device_id=peer, device_id_type=pl.DeviceIdType.MESH)
        ra = pltpu.make_async_remote_copy(
            src_ref=a_ref, dst_ref=aall.at[pl.ds(my_off, T)],
            send_sem=sem_sa, recv_sem=sem_ra,
            device_id=peer, device_id_type=pl.DeviceIdType.MESH)
        rx.start()
        ra.start()
        xall[pl.ds(my_off, T), :] = x_ref[...]
        aall[pl.ds(my_off, T), :] = a_ref[...]
        rx.wait()
        ra.wait()

        for e in range(E):
            slot = e % 2
            wcp[e][0].wait()
            wcp[e][1].wait()
            gid = my_x * E + e
            for c in range(TT // CHUNK):
                r0 = c * CHUNK
                xa = xall[pl.ds(r0, CHUNK), :]
                h = jnp.maximum(
                    jnp.dot(xa, w1v[slot],
                            preferred_element_type=jnp.float32),
                    0.0).astype(jnp.bfloat16)
                y = jnp.dot(h, w2v[slot],
                            preferred_element_type=jnp.float32)
                m = aall[pl.ds(r0, CHUNK), :] == gid
                contrib = jnp.where(m, y, 0.0).astype(jnp.bfloat16)
                if e == 0:
                    acc[pl.ds(r0, CHUNK), :] = contrib
                else:
                    acc[pl.ds(r0, CHUNK), :] = acc[pl.ds(r0, CHUNK), :] + contrib
            if e + 1 < E:
                wcp[e + 1][0].start()
                wcp[e + 1][1].start()

        rr = pltpu.make_async_remote_copy(
            src_ref=acc.at[pl.ds((1 - my_x) * T, T)], dst_ref=resbuf,
            send_sem=sem_sr, recv_sem=sem_rr,
            device_id=peer, device_id_type=pl.DeviceIdType.MESH)
        rr.start()
        rr.wait()
        out_ref[...] = (acc[pl.ds(my_off, T), :].astype(jnp.float32)
                        + resbuf[...].astype(jnp.float32))

    return pl.pallas_call(
        body,
        out_shape=jax.ShapeDtypeStruct((T, D), jnp.float32),
        in_specs=[
            pl.BlockSpec(memory_space=pltpu.VMEM),
            pl.BlockSpec(memory_space=pltpu.VMEM),
            pl.BlockSpec(memory_space=pltpu.ANY),
            pl.BlockSpec(memory_space=pltpu.ANY),
        ],
        out_specs=pl.BlockSpec(memory_space=pltpu.VMEM),
        scratch_shapes=[
            pltpu.VMEM((TT, D), jnp.bfloat16),
            pltpu.VMEM((TT, 1), jnp.int32),
            pltpu.VMEM((TT, D), jnp.bfloat16),
            pltpu.VMEM((T, D), jnp.bfloat16),
            pltpu.VMEM((2, D, F), jnp.bfloat16),
            pltpu.VMEM((2, F, D), jnp.bfloat16),
            pltpu.SemaphoreType.DMA,
            pltpu.SemaphoreType.DMA,
            pltpu.SemaphoreType.DMA,
            pltpu.SemaphoreType.DMA,
            pltpu.SemaphoreType.DMA,
            pltpu.SemaphoreType.DMA,
            pltpu.SemaphoreType.DMA((2,)),
            pltpu.SemaphoreType.DMA((2,)),
        ],
        compiler_params=pltpu.CompilerParams(collective_id=0),
    )(xb, a2, w1b, w2b)


# baseline (device time: 319332 ns/iter reference)
import jax
import jax.numpy as jnp
from jax import lax
from jax.experimental import pallas as pl
from jax.experimental.pallas import tpu as pltpu

NX = 2
CHUNK = 512


def kernel(x, assign, W1, W2):
    T, D = x.shape
    E, _, F = W1.shape
    TT = NX * T

    xb = x.astype(jnp.bfloat16)
    a2 = assign.reshape(T, 1)
    w1b = W1.astype(jnp.bfloat16)
    w2b = W2.astype(jnp.bfloat16)

    def body(x_ref, a_ref, w1_hbm, w2_hbm, out_ref,
             xall, aall, acc, resbuf, w1v, w2v,
             sem_sx, sem_rx, sem_sa, sem_ra, sem_sr, sem_rr,
             w1sem, w2sem):
        my_x = lax.axis_index("x")
        my_y = lax.axis_index("y")
        my_z = lax.axis_index("z")
        peer = (1 - my_x, my_y, my_z)

        wcp = [
            (pltpu.make_async_copy(w1_hbm.at[e], w1v.at[e % 2], w1sem.at[e % 2]),
             pltpu.make_async_copy(w2_hbm.at[e], w2v.at[e % 2], w2sem.at[e % 2]))
            for e in range(E)
        ]
        wcp[0][0].start()
        wcp[0][1].start()

        barrier = pltpu.get_barrier_semaphore()
        pl.semaphore_signal(barrier, inc=1, device_id=peer,
                            device_id_type=pl.DeviceIdType.MESH)
        pl.semaphore_wait(barrier, 1)

        my_off = my_x * T
        rx = pltpu.make_async_remote_copy(
            src_ref=x_ref, dst_ref=xall.at[pl.ds(my_off, T)],
            send_sem=sem_sx, recv_sem=sem_rx,
            device_id=peer, device_id_type=pl.DeviceIdType.MESH)
        ra = pltpu.make_async_remote_copy(
            src_ref=a_ref, dst_ref=aall.at[pl.ds(my_off, T)],
            send_sem=sem_sa, recv_sem=sem_ra,
            device_id=peer, device_id_type=pl.DeviceIdType.MESH)
        rx.start()
        ra.start()
        xall[pl.ds(my_off, T), :] = x_ref[...]
        aall[pl.ds(my_off, T), :] = a_ref[...]
        rx.wait()
        ra.wait()

        for e in range(E):
            slot = e % 2
            wcp[e][0].wait()
            wcp[e][1].wait()
            gid = my_x * E + e

            def chunk_step(c, _, slot=slot, first=(e == 0), gid=gid):
                r0 = c * CHUNK
                xa = xall[pl.ds(r0, CHUNK), :]
                h = jnp.maximum(
                    jnp.dot(xa, w1v[slot],
                            preferred_element_type=jnp.float32),
                    0.0).astype(jnp.bfloat16)
                y = jnp.dot(h, w2v[slot],
                            preferred_element_type=jnp.float32)
                m = aall[pl.ds(r0, CHUNK), :] == gid
                contrib = jnp.where(m, y, 0.0).astype(jnp.bfloat16)
                if first:
                    acc[pl.ds(r0, CHUNK), :] = contrib
                else:
                    acc[pl.ds(r0, CHUNK), :] = acc[pl.ds(r0, CHUNK), :] + contrib
                return 0

            lax.fori_loop(0, TT // CHUNK, chunk_step, 0)
            if e + 1 < E:
                wcp[e + 1][0].start()
                wcp[e + 1][1].start()

        rr = pltpu.make_async_remote_copy(
            src_ref=acc.at[pl.ds((1 - my_x) * T, T)], dst_ref=resbuf,
            send_sem=sem_sr, recv_sem=sem_rr,
            device_id=peer, device_id_type=pl.DeviceIdType.MESH)
        rr.start()
        rr.wait()
        out_ref[...] = (acc[pl.ds(my_off, T), :].astype(jnp.float32)
                        + resbuf[...].astype(jnp.float32))

    return pl.pallas_call(
        body,
        out_shape=jax.ShapeDtypeStruct((T, D), jnp.float32),
        in_specs=[
            pl.BlockSpec(memory_space=pltpu.VMEM),
            pl.BlockSpec(memory_space=pltpu.VMEM),
            pl.BlockSpec(memory_space=pltpu.MemorySpace.HBM),
            pl.BlockSpec(memory_space=pltpu.MemorySpace.HBM),
        ],
        out_specs=pl.BlockSpec(memory_space=pltpu.VMEM),
        scratch_shapes=[
            pltpu.VMEM((TT, D), jnp.bfloat16),
            pltpu.VMEM((TT, 1), jnp.int32),
            pltpu.VMEM((TT, D), jnp.bfloat16),
            pltpu.VMEM((T, D), jnp.bfloat16),
            pltpu.VMEM((2, D, F), jnp.bfloat16),
            pltpu.VMEM((2, F, D), jnp.bfloat16),
            pltpu.SemaphoreType.DMA,
            pltpu.SemaphoreType.DMA,
            pltpu.SemaphoreType.DMA,
            pltpu.SemaphoreType.DMA,
            pltpu.SemaphoreType.DMA,
            pltpu.SemaphoreType.DMA,
            pltpu.SemaphoreType.DMA((2,)),
            pltpu.SemaphoreType.DMA((2,)),
        ],
        compiler_params=pltpu.CompilerParams(
            collective_id=0, vmem_limit_bytes=100 * 1024 * 1024),
    )(xb, a2, w1b, w2b)


# device time: 217194 ns/iter; 1.4703x vs baseline; 1.4703x over previous
import jax
import jax.numpy as jnp
from jax import lax
from jax.experimental import pallas as pl
from jax.experimental.pallas import tpu as pltpu

NX = 2
NB = 4
MESH = None


def kernel(x, assign, W1, W2):
    T, D = x.shape
    E, _, F = W1.shape
    BLK = T // NB

    xb = x.astype(jnp.bfloat16)
    a2 = assign.reshape(T, 1)
    w1b = W1.astype(jnp.bfloat16)
    w2b = W2.astype(jnp.bfloat16)

    def body(x_ref, a_ref, w1_ref, w2_ref, out_ref,
             xpeer, apeer, accpeer, resbuf,
             sxd, rxd, sad, rad, scb, rcb):
        my_x = lax.axis_index("x")
        my_y = lax.axis_index("y")
        my_z = lax.axis_index("z")
        peer = (1 - my_x, my_y, my_z)

        def disp_rdma(b):
            return pltpu.make_async_remote_copy(
                src_ref=x_ref.at[pl.ds(b * BLK, BLK)],
                dst_ref=xpeer.at[pl.ds(b * BLK, BLK)],
                send_sem=sxd.at[b], recv_sem=rxd.at[b],
                device_id=peer, device_id_type=pl.DeviceIdType.MESH)

        def comb_rdma(b):
            return pltpu.make_async_remote_copy(
                src_ref=accpeer.at[pl.ds(b * BLK, BLK)],
                dst_ref=resbuf.at[pl.ds(b * BLK, BLK)],
                send_sem=scb.at[b], recv_sem=rcb.at[b],
                device_id=peer, device_id_type=pl.DeviceIdType.MESH)

        ra = pltpu.make_async_remote_copy(
            src_ref=a_ref, dst_ref=apeer, send_sem=sad, recv_sem=rad,
            device_id=peer, device_id_type=pl.DeviceIdType.MESH)

        barrier = pltpu.get_barrier_semaphore()
        pl.semaphore_signal(barrier, inc=1, device_id=peer,
                            device_id_type=pl.DeviceIdType.MESH)
        pl.semaphore_wait(barrier, 1)

        def start_disp(b, c):
            disp_rdma(b).start()
            return c
        lax.fori_loop(0, NB, start_disp, 0)
        ra.start()

        def block_accum(xa, ablk):
            yacc = None
            for e in range(E):
                gid = my_x * E + e
                xm = jnp.where(ablk == gid, xa, 0)
                h = jnp.maximum(
                    jnp.dot(xm, w1_ref[e],
                            preferred_element_type=jnp.float32),
                    0.0).astype(jnp.bfloat16)
                y = jnp.dot(h, w2_ref[e], preferred_element_type=jnp.float32)
                yacc = y if yacc is None else yacc + y
            return yacc.astype(jnp.bfloat16)

        def my_block(b, c):
            r0 = b * BLK
            out_ref[pl.ds(r0, BLK), :] = block_accum(
                x_ref[pl.ds(r0, BLK), :], a_ref[pl.ds(r0, BLK), :])
            return c
        lax.fori_loop(0, NB, my_block, 0)

        ra.wait()

        def peer_block(b, c):
            disp_rdma(b).wait_recv()
            r0 = b * BLK
            accpeer[pl.ds(r0, BLK), :] = block_accum(
                xpeer[pl.ds(r0, BLK), :], apeer[pl.ds(r0, BLK), :])
            comb_rdma(b).start()
            return c
        lax.fori_loop(0, NB, peer_block, 0)

        def drain(b, c):
            disp_rdma(b).wait_send()
            comb_rdma(b).wait()
            return c
        lax.fori_loop(0, NB, drain, 0)

        out_ref[...] = out_ref[...] + resbuf[...]

    return pl.pallas_call(
        body,
        out_shape=jax.ShapeDtypeStruct((T, D), jnp.bfloat16),
        in_specs=[
            pl.BlockSpec(memory_space=pltpu.VMEM),
            pl.BlockSpec(memory_space=pltpu.VMEM),
            pl.BlockSpec(memory_space=pltpu.VMEM),
            pl.BlockSpec(memory_space=pltpu.VMEM),
        ],
        out_specs=pl.BlockSpec(memory_space=pltpu.VMEM),
        scratch_shapes=[
            pltpu.VMEM((T, D), jnp.bfloat16),
            pltpu.VMEM((T, 1), jnp.int32),
            pltpu.VMEM((T, D), jnp.bfloat16),
            pltpu.VMEM((T, D), jnp.bfloat16),
            pltpu.SemaphoreType.DMA((NB,)),
            pltpu.SemaphoreType.DMA((NB,)),
            pltpu.SemaphoreType.DMA,
            pltpu.SemaphoreType.DMA,
            pltpu.SemaphoreType.DMA((NB,)),
            pltpu.SemaphoreType.DMA((NB,)),
        ],
        compiler_params=pltpu.CompilerParams(
            collective_id=0, vmem_limit_bytes=100 * 1024 * 1024),
    )(xb, a2, w1b, w2b)


# device time: 193339 ns/iter; 1.6517x vs baseline; 1.1234x over previous
import jax
import jax.numpy as jnp
from jax import lax
from jax.experimental import pallas as pl
from jax.experimental.pallas import tpu as pltpu

NX = 2
NBH = 4
NBT = 2 * NBH
EPY = 2


def kernel(x, assign, W1, W2):
    T, D = x.shape
    E, _, F = W1.shape
    TT = NX * T
    BLK = T // NBH

    xb = x.astype(jnp.bfloat16)
    a2 = assign.reshape(T, 1)
    w1b = W1.astype(jnp.bfloat16)
    w2b = W2.astype(jnp.bfloat16)

    def body(x_ref, a_ref, w1_hbm, w2_hbm, out_ref,
             xpeer, apeer, C, yrecv, resbuf, wv1, wv2,
             sxd, rxd, sad, rad, syd, ryd, scb, rcb, w1sem, w2sem):
        my_x = lax.axis_index("x")
        my_y = lax.axis_index("y")
        my_z = lax.axis_index("z")
        xpeer_id = (1 - my_x, my_y, my_z)
        ypeer_id = (my_x, 1 - my_y, my_z)

        wc = [pltpu.make_async_copy(w1_hbm.at[EPY * my_y + j], wv1.at[j],
                                    w1sem.at[j]) for j in range(EPY)]
        wc += [pltpu.make_async_copy(w2_hbm.at[EPY * my_y + j], wv2.at[j],
                                     w2sem.at[j]) for j in range(EPY)]
        for c in wc:
            c.start()

        barrier = pltpu.get_barrier_semaphore()
        for nbr in (xpeer_id, ypeer_id):
            pl.semaphore_signal(barrier, inc=1, device_id=nbr,
                                device_id_type=pl.DeviceIdType.MESH)
        pl.semaphore_wait(barrier, 2)

        def disp_rdma(b):
            return pltpu.make_async_remote_copy(
                src_ref=x_ref.at[pl.ds(b * BLK, BLK)],
                dst_ref=xpeer.at[pl.ds(b * BLK, BLK)],
                send_sem=sxd.at[b], recv_sem=rxd.at[b],
                device_id=xpeer_id, device_id_type=pl.DeviceIdType.MESH)

        def ysend_rdma(g):
            return pltpu.make_async_remote_copy(
                src_ref=C.at[pl.ds(g * BLK, BLK)],
                dst_ref=yrecv.at[pl.ds(g * BLK, BLK)],
                send_sem=syd.at[g], recv_sem=ryd.at[g],
                device_id=ypeer_id, device_id_type=pl.DeviceIdType.MESH)

        def xcomb_rdma(b):
            g = (1 - my_x) * NBH + b
            return pltpu.make_async_remote_copy(
                src_ref=C.at[pl.ds(g * BLK, BLK)],
                dst_ref=resbuf.at[pl.ds(b * BLK, BLK)],
                send_sem=scb.at[b], recv_sem=rcb.at[b],
                device_id=xpeer_id, device_id_type=pl.DeviceIdType.MESH)

        ra = pltpu.make_async_remote_copy(
            src_ref=a_ref, dst_ref=apeer, send_sem=sad, recv_sem=rad,
            device_id=xpeer_id, device_id_type=pl.DeviceIdType.MESH)

        def start_disp(b, c):
            disp_rdma(b).start()
            return c
        lax.fori_loop(0, NBH, start_disp, 0)
        ra.start()

        for c in wc:
            c.wait()

        def block_accum(xa, ablk):
            yacc = None
            for j in range(EPY):
                gid = my_x * E + EPY * my_y + j
                xm = jnp.where(ablk == gid, xa, 0)
                h = jnp.maximum(
                    jnp.dot(xm, wv1[j], preferred_element_type=jnp.float32),
                    0.0).astype(jnp.bfloat16)
                y = jnp.dot(h, wv2[j], preferred_element_type=jnp.float32)
                yacc = y if yacc is None else yacc + y
            return yacc.astype(jnp.bfloat16)

        def my_block(b, c):
            g = my_x * NBH + b
            C[pl.ds(g * BLK, BLK), :] = block_accum(
                x_ref[pl.ds(b * BLK, BLK), :], a_ref[pl.ds(b * BLK, BLK), :])
            ysend_rdma(g).start()
            return c
        lax.fori_loop(0, NBH, my_block, 0)

        ra.wait()

        def peer_block(b, c):
            disp_rdma(b).wait_recv()
            g = (1 - my_x) * NBH + b
            C[pl.ds(g * BLK, BLK), :] = block_accum(
                xpeer[pl.ds(b * BLK, BLK), :], apeer[pl.ds(b * BLK, BLK), :])
            ysend_rdma(g).start()
            return c
        lax.fori_loop(0, NBH, peer_block, 0)

        def yadd_peer(b, c):
            g = (1 - my_x) * NBH + b
            r = ysend_rdma(g)
            r.wait_send()
            r.wait_recv()
            C[pl.ds(g * BLK, BLK), :] = (C[pl.ds(g * BLK, BLK), :]
                                         + yrecv[pl.ds(g * BLK, BLK), :])
            xcomb_rdma(b).start()
            return c
        lax.fori_loop(0, NBH, yadd_peer, 0)

        def yadd_mine(b, c):
            g = my_x * NBH + b
            r = ysend_rdma(g)
            r.wait_send()
            r.wait_recv()
            C[pl.ds(g * BLK, BLK), :] = (C[pl.ds(g * BLK, BLK), :]
                                         + yrecv[pl.ds(g * BLK, BLK), :])
            return c
        lax.fori_loop(0, NBH, yadd_mine, 0)

        def finish(b, c):
            xcomb_rdma(b).wait_recv()
            g = my_x * NBH + b
            out_ref[pl.ds(b * BLK, BLK), :] = (
                C[pl.ds(g * BLK, BLK), :] + resbuf[pl.ds(b * BLK, BLK), :])
            return c
        lax.fori_loop(0, NBH, finish, 0)

        def drain(b, c):
            disp_rdma(b).wait_send()
            xcomb_rdma(b).wait_send()
            return c
        lax.fori_loop(0, NBH, drain, 0)

    return pl.pallas_call(
        body,
        out_shape=jax.ShapeDtypeStruct((T, D), jnp.bfloat16),
        in_specs=[
            pl.BlockSpec(memory_space=pltpu.VMEM),
            pl.BlockSpec(memory_space=pltpu.VMEM),
            pl.BlockSpec(memory_space=pltpu.MemorySpace.HBM),
            pl.BlockSpec(memory_space=pltpu.MemorySpace.HBM),
        ],
        out_specs=pl.BlockSpec(memory_space=pltpu.VMEM),
        scratch_shapes=[
            pltpu.VMEM((T, D), jnp.bfloat16),
            pltpu.VMEM((T, 1), jnp.int32),
            pltpu.VMEM((TT, D), jnp.bfloat16),
            pltpu.VMEM((TT, D), jnp.bfloat16),
            pltpu.VMEM((T, D), jnp.bfloat16),
            pltpu.VMEM((EPY, D, F), jnp.bfloat16),
            pltpu.VMEM((EPY, F, D), jnp.bfloat16),
            pltpu.SemaphoreType.DMA((NBH,)),
            pltpu.SemaphoreType.DMA((NBH,)),
            pltpu.SemaphoreType.DMA,
            pltpu.SemaphoreType.DMA,
            pltpu.SemaphoreType.DMA((NBT,)),
            pltpu.SemaphoreType.DMA((NBT,)),
            pltpu.SemaphoreType.DMA((NBH,)),
            pltpu.SemaphoreType.DMA((NBH,)),
            pltpu.SemaphoreType.DMA((EPY,)),
            pltpu.SemaphoreType.DMA((EPY,)),
        ],
        compiler_params=pltpu.CompilerParams(
            collective_id=0, vmem_limit_bytes=100 * 1024 * 1024),
    )(xb, a2, w1b, w2b)


# device time: 189925 ns/iter; 1.6814x vs baseline; 1.0180x over previous
import jax
import jax.numpy as jnp
from jax import lax
from jax.experimental import pallas as pl
from jax.experimental.pallas import tpu as pltpu

NX = 2
NBH = 4
EPY = 2


def kernel(x, assign, W1, W2):
    T, D = x.shape
    E, _, F = W1.shape
    BLK = T // NBH

    xb = x.astype(jnp.bfloat16)
    a2 = assign.reshape(T, 1)
    my_y0 = EPY * lax.axis_index("y")
    w1b = lax.dynamic_slice_in_dim(W1, my_y0, EPY, 0).astype(jnp.bfloat16)
    w2b = lax.dynamic_slice_in_dim(W2, my_y0, EPY, 0).astype(jnp.bfloat16)

    def body(x_ref, a_ref, w1_hbm, w2_hbm, out_ref,
             xpeer, apeer, C, yrecv, resbuf, wv1, wv2,
             sxd, rxd, sad, rad, sxp, rxp, syd, ryd, w1sem, w2sem):
        my_x = lax.axis_index("x")
        my_y = lax.axis_index("y")
        my_z = lax.axis_index("z")
        xpeer_id = (1 - my_x, my_y, my_z)
        ypeer_id = (my_x, 1 - my_y, my_z)

        wc = [pltpu.make_async_copy(w1_hbm.at[j], wv1.at[j], w1sem.at[j])
              for j in range(EPY)]
        wc += [pltpu.make_async_copy(w2_hbm.at[j], wv2.at[j], w2sem.at[j])
               for j in range(EPY)]
        for c in wc:
            c.start()

        barrier = pltpu.get_barrier_semaphore()
        for nbr in (xpeer_id, ypeer_id):
            pl.semaphore_signal(barrier, inc=1, device_id=nbr,
                                device_id_type=pl.DeviceIdType.MESH)
        pl.semaphore_wait(barrier, 2)

        def disp_rdma(b):
            return pltpu.make_async_remote_copy(
                src_ref=x_ref.at[pl.ds(b * BLK, BLK)],
                dst_ref=xpeer.at[pl.ds(b * BLK, BLK)],
                send_sem=sxd.at[b], recv_sem=rxd.at[b],
                device_id=xpeer_id, device_id_type=pl.DeviceIdType.MESH)

        def xpart_rdma(b):
            g = (1 - my_x) * NBH + b
            return pltpu.make_async_remote_copy(
                src_ref=C.at[pl.ds(g * BLK, BLK)],
                dst_ref=resbuf.at[pl.ds(b * BLK, BLK)],
                send_sem=sxp.at[b], recv_sem=rxp.at[b],
                device_id=xpeer_id, device_id_type=pl.DeviceIdType.MESH)

        def ysend_rdma(b):
            g = my_x * NBH + b
            return pltpu.make_async_remote_copy(
                src_ref=C.at[pl.ds(g * BLK, BLK)],
                dst_ref=yrecv.at[pl.ds(b * BLK, BLK)],
                send_sem=syd.at[b], recv_sem=ryd.at[b],
                device_id=ypeer_id, device_id_type=pl.DeviceIdType.MESH)

        ra = pltpu.make_async_remote_copy(
            src_ref=a_ref, dst_ref=apeer, send_sem=sad, recv_sem=rad,
            device_id=xpeer_id, device_id_type=pl.DeviceIdType.MESH)

        def start_disp(b, c):
            disp_rdma(b).start()
            return c
        lax.fori_loop(0, NBH, start_disp, 0)
        ra.start()

        for c in wc:
            c.wait()

        def block_accum(xa, ablk):
            yacc = None
            for j in range(EPY):
                gid = my_x * E + EPY * my_y + j
                xm = jnp.where(ablk == gid, xa, 0)
                h = jnp.maximum(
                    jnp.dot(xm, wv1[j], preferred_element_type=jnp.float32),
                    0.0).astype(jnp.bfloat16)
                y = jnp.dot(h, wv2[j], preferred_element_type=jnp.float32)
                yacc = y if yacc is None else yacc + y
            return yacc.astype(jnp.bfloat16)

        ra.wait()

        def peer_block(b, c):
            disp_rdma(b).wait_recv()
            g = (1 - my_x) * NBH + b
            C[pl.ds(g * BLK, BLK), :] = block_accum(
                xpeer[pl.ds(b * BLK, BLK), :], apeer[pl.ds(b * BLK, BLK), :])
            xpart_rdma(b).start()
            return c
        lax.fori_loop(0, NBH, peer_block, 0)

        def my_block(b, c):
            g = my_x * NBH + b
            C[pl.ds(g * BLK, BLK), :] = block_accum(
                x_ref[pl.ds(b * BLK, BLK), :], a_ref[pl.ds(b * BLK, BLK), :])
            xpart_rdma(b).wait_recv()
            C[pl.ds(g * BLK, BLK), :] = (C[pl.ds(g * BLK, BLK), :]
                                         + resbuf[pl.ds(b * BLK, BLK), :])
            ysend_rdma(b).start()
            return c
        lax.fori_loop(0, NBH, my_block, 0)

        def finish(b, c):
            r = ysend_rdma(b)
            r.wait_recv()
            g = my_x * NBH + b
            out_ref[pl.ds(b * BLK, BLK), :] = (
                C[pl.ds(g * BLK, BLK), :] + yrecv[pl.ds(b * BLK, BLK), :])
            return c
        lax.fori_loop(0, NBH, finish, 0)

        def drain(b, c):
            disp_rdma(b).wait_send()
            xpart_rdma(b).wait_send()
            ysend_rdma(b).wait_send()
            return c
        lax.fori_loop(0, NBH, drain, 0)

    return pl.pallas_call(
        body,
        out_shape=jax.ShapeDtypeStruct((T, D), jnp.bfloat16),
        in_specs=[
            pl.BlockSpec(memory_space=pltpu.VMEM),
            pl.BlockSpec(memory_space=pltpu.VMEM),
            pl.BlockSpec(memory_space=pltpu.MemorySpace.HBM),
            pl.BlockSpec(memory_space=pltpu.MemorySpace.HBM),
        ],
        out_specs=pl.BlockSpec(memory_space=pltpu.VMEM),
        scratch_shapes=[
            pltpu.VMEM((T, D), jnp.bfloat16),
            pltpu.VMEM((T, 1), jnp.int32),
            pltpu.VMEM((NX * T, D), jnp.bfloat16),
            pltpu.VMEM((T, D), jnp.bfloat16),
            pltpu.VMEM((T, D), jnp.bfloat16),
            pltpu.VMEM((EPY, D, F), jnp.bfloat16),
            pltpu.VMEM((EPY, F, D), jnp.bfloat16),
            pltpu.SemaphoreType.DMA((NBH,)),
            pltpu.SemaphoreType.DMA((NBH,)),
            pltpu.SemaphoreType.DMA,
            pltpu.SemaphoreType.DMA,
            pltpu.SemaphoreType.DMA((NBH,)),
            pltpu.SemaphoreType.DMA((NBH,)),
            pltpu.SemaphoreType.DMA((NBH,)),
            pltpu.SemaphoreType.DMA((NBH,)),
            pltpu.SemaphoreType.DMA((EPY,)),
            pltpu.SemaphoreType.DMA((EPY,)),
        ],
        compiler_params=pltpu.CompilerParams(
            collective_id=0, vmem_limit_bytes=100 * 1024 * 1024),
    )(xb, a2, w1b, w2b)


# device time: 180504 ns/iter; 1.7691x vs baseline; 1.0522x over previous
import jax
import jax.numpy as jnp
from jax import lax
from jax.experimental import pallas as pl
from jax.experimental.pallas import tpu as pltpu

NX = 2
NBH = 4
EPY = 2


def kernel(x, assign, W1, W2):
    T, D = x.shape
    E, _, F = W1.shape
    BLK = T // NBH

    xb = x.astype(jnp.bfloat16)
    a2 = assign.reshape(T, 1)
    my_y0 = EPY * lax.axis_index("y")
    w1b = lax.dynamic_slice_in_dim(W1, my_y0, EPY, 0).astype(jnp.bfloat16)
    w2b = lax.dynamic_slice_in_dim(W2, my_y0, EPY, 0).astype(jnp.bfloat16)

    def body(x_ref, a_ref, w1_hbm, w2_hbm, out_ref,
             xpeer, apeer, C, yrecv, resbuf, wv1, wv2,
             sxd, rxd, sad, rad, sxp, rxp, syd, ryd, w1sem, w2sem):
        my_x = lax.axis_index("x")
        my_y = lax.axis_index("y")
        my_z = lax.axis_index("z")
        xpeer_id = (1 - my_x, my_y, my_z)
        ypeer_id = (my_x, 1 - my_y, my_z)

        wc = [pltpu.make_async_copy(w1_hbm.at[j], wv1.at[j], w1sem.at[j])
              for j in range(EPY)]
        wc += [pltpu.make_async_copy(w2_hbm.at[j], wv2.at[j], w2sem.at[j])
               for j in range(EPY)]
        for c in wc:
            c.start()

        barrier = pltpu.get_barrier_semaphore()
        for nbr in (xpeer_id, ypeer_id):
            pl.semaphore_signal(barrier, inc=1, device_id=nbr,
                                device_id_type=pl.DeviceIdType.MESH)
        pl.semaphore_wait(barrier, 2)

        def disp_rdma(b):
            return pltpu.make_async_remote_copy(
                src_ref=x_ref.at[pl.ds(b * BLK, BLK)],
                dst_ref=xpeer.at[pl.ds(b * BLK, BLK)],
                send_sem=sxd.at[b], recv_sem=rxd.at[b],
                device_id=xpeer_id, device_id_type=pl.DeviceIdType.MESH)

        def xpart_rdma(b):
            g = (1 - my_x) * NBH + b
            return pltpu.make_async_remote_copy(
                src_ref=C.at[pl.ds(g * BLK, BLK)],
                dst_ref=resbuf.at[pl.ds(b * BLK, BLK)],
                send_sem=sxp.at[b], recv_sem=rxp.at[b],
                device_id=xpeer_id, device_id_type=pl.DeviceIdType.MESH)

        def ysend_rdma(b):
            g = my_x * NBH + b
            return pltpu.make_async_remote_copy(
                src_ref=C.at[pl.ds(g * BLK, BLK)],
                dst_ref=yrecv.at[pl.ds(b * BLK, BLK)],
                send_sem=syd.at[b], recv_sem=ryd.at[b],
                device_id=ypeer_id, device_id_type=pl.DeviceIdType.MESH)

        ra = pltpu.make_async_remote_copy(
            src_ref=a_ref, dst_ref=apeer, send_sem=sad, recv_sem=rad,
            device_id=xpeer_id, device_id_type=pl.DeviceIdType.MESH)

        def start_disp(b, c):
            disp_rdma(b).start()
            return c
        lax.fori_loop(0, NBH, start_disp, 0)
        ra.start()

        for c in wc:
            c.wait()

        def block_accum(xa, ablk):
            yacc = None
            for j in range(EPY):
                gid = my_x * E + EPY * my_y + j
                xm = jnp.where(ablk == gid, xa, 0)
                h = jnp.maximum(
                    jnp.dot(xm, wv1[j], preferred_element_type=jnp.float32),
                    0.0).astype(jnp.bfloat16)
                y = jnp.dot(h, wv2[j], preferred_element_type=jnp.float32)
                yacc = y if yacc is None else yacc + y
            return yacc.astype(jnp.bfloat16)

        def my_block(b, c):
            g = my_x * NBH + b
            C[pl.ds(g * BLK, BLK), :] = block_accum(
                x_ref[pl.ds(b * BLK, BLK), :], a_ref[pl.ds(b * BLK, BLK), :])
            return c
        lax.fori_loop(0, NBH, my_block, 0)

        ra.wait()

        def peer_block(b, c):
            disp_rdma(b).wait_recv()
            g = (1 - my_x) * NBH + b
            C[pl.ds(g * BLK, BLK), :] = block_accum(
                xpeer[pl.ds(b * BLK, BLK), :], apeer[pl.ds(b * BLK, BLK), :])
            xpart_rdma(b).start()
            return c
        lax.fori_loop(0, NBH, peer_block, 0)

        def pre_add(b, c):
            g = my_x * NBH + b
            xpart_rdma(b).wait_recv()
            C[pl.ds(g * BLK, BLK), :] = (C[pl.ds(g * BLK, BLK), :]
                                         + resbuf[pl.ds(b * BLK, BLK), :])
            ysend_rdma(b).start()
            return c
        lax.fori_loop(0, NBH, pre_add, 0)

        def finish(b, c):
            r = ysend_rdma(b)
            r.wait_recv()
            g = my_x * NBH + b
            out_ref[pl.ds(b * BLK, BLK), :] = (
                C[pl.ds(g * BLK, BLK), :] + yrecv[pl.ds(b * BLK, BLK), :])
            return c
        lax.fori_loop(0, NBH, finish, 0)

        def drain(b, c):
            disp_rdma(b).wait_send()
            xpart_rdma(b).wait_send()
            ysend_rdma(b).wait_send()
            return c
        lax.fori_loop(0, NBH, drain, 0)

    return pl.pallas_call(
        body,
        out_shape=jax.ShapeDtypeStruct((T, D), jnp.bfloat16),
        in_specs=[
            pl.BlockSpec(memory_space=pltpu.VMEM),
            pl.BlockSpec(memory_space=pltpu.VMEM),
            pl.BlockSpec(memory_space=pltpu.MemorySpace.HBM),
            pl.BlockSpec(memory_space=pltpu.MemorySpace.HBM),
        ],
        out_specs=pl.BlockSpec(memory_space=pltpu.VMEM),
        scratch_shapes=[
            pltpu.VMEM((T, D), jnp.bfloat16),
            pltpu.VMEM((T, 1), jnp.int32),
            pltpu.VMEM((NX * T, D), jnp.bfloat16),
            pltpu.VMEM((T, D), jnp.bfloat16),
            pltpu.VMEM((T, D), jnp.bfloat16),
            pltpu.VMEM((EPY, D, F), jnp.bfloat16),
            pltpu.VMEM((EPY, F, D), jnp.bfloat16),
            pltpu.SemaphoreType.DMA((NBH,)),
            pltpu.SemaphoreType.DMA((NBH,)),
            pltpu.SemaphoreType.DMA,
            pltpu.SemaphoreType.DMA,
            pltpu.SemaphoreType.DMA((NBH,)),
            pltpu.SemaphoreType.DMA((NBH,)),
            pltpu.SemaphoreType.DMA((NBH,)),
            pltpu.SemaphoreType.DMA((NBH,)),
            pltpu.SemaphoreType.DMA((EPY,)),
            pltpu.SemaphoreType.DMA((EPY,)),
        ],
        compiler_params=pltpu.CompilerParams(
            collective_id=0, vmem_limit_bytes=100 * 1024 * 1024),
    )(xb, a2, w1b, w2b)
